# baseline (speedup 1.0000x reference)
import functools

import jax
import jax.numpy as jnp
from jax import lax
from jax.experimental import pallas as pl
from jax.experimental.pallas import tpu as pltpu
from jax.experimental.pallas import tpu_sc as plsc

CUT0, CUT1 = 20000, 200000
D0, D1, D2 = 128, 32, 8
ED = 128
LANES = 16
CH = 640
V2 = 800000
M0, M1, M2 = 16383, 131071, 524287


def _sc_gather(xf, t0, t1, t2flat):
    n = xf.shape[0]
    info = plsc.get_sparse_core_info()
    nw = info.num_cores * info.num_subcores
    per_w = n // nw
    n_chunks = per_w // CH
    assert per_w % CH == 0 and n % nw == 0
    n4 = n // 4
    w_per_q = nw // 4

    mesh = plsc.VectorSubcoreMesh(core_axis_name="c", subcore_axis_name="s")

    @functools.partial(
        pl.kernel,
        out_type=(
            jax.ShapeDtypeStruct((n, D0), jnp.float32),
            jax.ShapeDtypeStruct((n4, 128), jnp.float32),
            jax.ShapeDtypeStruct((D2, n), jnp.float32),
        ),
        mesh=mesh,
        compiler_params=pltpu.CompilerParams(use_tc_tiling_on_sc=False),
        scratch_types=[
            pltpu.VMEM((CH,), jnp.int32),
            pltpu.VMEM((CH,), jnp.int32),
            pltpu.VMEM((CH,), jnp.int32),
            pltpu.VMEM((D2 * CH,), jnp.int32),
            pltpu.VMEM((CH, D0), jnp.float32),
            pltpu.VMEM((CH, D1), jnp.float32),
            pltpu.VMEM((D2 * CH,), jnp.float32),
            pltpu.SemaphoreType.DMA,
            pltpu.SemaphoreType.DMA,
            pltpu.SemaphoreType.DMA,
        ],
    )
    def sc_kernel(x_hbm, t0_hbm, t1_hbm, t2f_hbm, r0_hbm, r1p_hbm, g2t_hbm,
                  x_v, i0_v, i1_v, i2_v, r0_v, r1_v, g2_v, sem0, sem1, sem2):
        wid = lax.axis_index("s") * info.num_cores + lax.axis_index("c")
        w_base = wid * per_w
        q = wid // w_per_q
        r1_row0 = (wid % w_per_q) * per_w
        c1_off = 32 * q

        def body(j, carry):
            base = w_base + j * CH
            pltpu.sync_copy(x_hbm.at[pl.ds(base, CH)], x_v)
            for i in range(CH // LANES):
                sl = pl.ds(i * LANES, LANES)
                xv = x_v[sl]
                zeros = jnp.zeros_like(xv)
                pv = base + i * LANES + lax.iota(jnp.int32, 16)
                i0_v[sl] = jnp.where(xv < CUT0, xv, pv & M0)
                d1 = pv & M1
                i1_v[sl] = jnp.where(
                    xv >= CUT0, jnp.where(xv < CUT1, xv - CUT0, d1), d1)
                i2 = jnp.where(xv >= CUT1, xv - CUT1, pv & M2)
                for c in range(D2):
                    i2_v[pl.ds(c * CH + i * LANES, LANES)] = i2 + c * V2
            c0 = pltpu.async_copy(t0_hbm.at[i0_v], r0_v, sem0)
            c1 = pltpu.async_copy(t1_hbm.at[i1_v], r1_v, sem1)
            c2 = pltpu.async_copy(t2f_hbm.at[i2_v], g2_v, sem2)
            c0.wait()
            c1.wait()
            c2.wait()
            pltpu.sync_copy(r0_v, r0_hbm.at[pl.ds(base, CH)])
            pltpu.sync_copy(
                r1_v, r1p_hbm.at[pl.ds(r1_row0 + j * CH, CH), pl.ds(c1_off, D1)])
            for c in range(D2):
                pltpu.sync_copy(g2_v.at[pl.ds(c * CH, CH)],
                                g2t_hbm.at[c, pl.ds(base, CH)])
            return carry

        lax.fori_loop(0, n_chunks, body, 0)

    return sc_kernel(xf, t0, t1, t2flat)


def _tc_project(cid, r0, r1p, g2t, w0, w1t, w2):
    n = r0.shape[0]
    bm = 512
    nb1 = (n // 4) // bm

    def body(cb, r0b, r1b, g2b, w0b, w1b, w2b, ob):
        q = pl.program_id(1)
        riota = lax.broadcasted_iota(jnp.int32, (128, 1), 0)
        w1sel = jnp.where((riota >= D1 * q) & (riota < D1 * q + D1),
                          w1b[...], 0.0)
        e0 = jnp.dot(r0b[...], w0b[...], preferred_element_type=jnp.float32)
        e1 = jnp.dot(r1b[...], w1sel, preferred_element_type=jnp.float32)
        e2 = lax.dot_general(
            g2b[...], w2b[...], (((0,), (0,)), ((), ())),
            preferred_element_type=jnp.float32)
        cv = cb[...].astype(jnp.int32)
        ob[...] = jnp.where(cv == 0, e0, jnp.where(cv == 1, e1, e2))

    return pl.pallas_call(
        body,
        grid=(nb1, 4),
        in_specs=[
            pl.BlockSpec((bm, 1), lambda i, j: (j * nb1 + i, 0)),
            pl.BlockSpec((bm, D0), lambda i, j: (j * nb1 + i, 0)),
            pl.BlockSpec((bm, 128), lambda i, j: (i, 0)),
            pl.BlockSpec((D2, bm), lambda i, j: (0, j * nb1 + i)),
            pl.BlockSpec((D0, ED), lambda i, j: (0, 0)),
            pl.BlockSpec((D0, ED), lambda i, j: (0, 0)),
            pl.BlockSpec((D2, ED), lambda i, j: (0, 0)),
        ],
        out_specs=pl.BlockSpec((bm, ED), lambda i, j: (j * nb1 + i, 0)),
        out_shape=jax.ShapeDtypeStruct((n, ED), jnp.float32),
    )(cid, r0, r1p, g2t, w0, w1t, w2)


def _sc_flatten_t2(t2T):
    d, v = t2T.shape
    info = plsc.get_sparse_core_info()
    nw = info.num_cores * info.num_subcores
    seg = v // (nw // d)
    mesh = plsc.VectorSubcoreMesh(core_axis_name="c", subcore_axis_name="s")

    @functools.partial(
        pl.kernel,
        out_type=jax.ShapeDtypeStruct((d * v,), jnp.float32),
        mesh=mesh,
        compiler_params=pltpu.CompilerParams(use_tc_tiling_on_sc=False),
    )
    def flat_kernel(src_hbm, dst_hbm):
        wid = lax.axis_index("s") * info.num_cores + lax.axis_index("c")
        c = wid // (nw // d)
        o = (wid % (nw // d)) * seg
        pltpu.sync_copy(src_hbm.at[c, pl.ds(o, seg)],
                        dst_hbm.at[pl.ds(c * v + o, seg)])

    return flat_kernel(t2T)


def kernel(x, t0, t1, t2, w0, w1, w2):
    b, s = x.shape
    n = b * s
    xf = x.reshape(n)
    cid = ((xf >= CUT0).astype(jnp.int8) + (xf >= CUT1).astype(jnp.int8))
    t2flat = _sc_flatten_t2(t2.T)
    r0, r1p, g2t = _sc_gather(xf, t0, t1, t2flat)
    w1t = jnp.tile(w1, (4, 1))
    out = _tc_project(cid.reshape(n, 1), r0, r1p, g2t, w0, w1t, w2)
    return out.reshape(b, s, ED)


# speedup vs baseline: 2.0850x; 2.0850x over previous
import functools

import jax
import jax.numpy as jnp
from jax import lax
from jax.experimental import pallas as pl
from jax.experimental.pallas import tpu as pltpu
from jax.experimental.pallas import tpu_sc as plsc

CUT0, CUT1 = 20000, 200000
D0, D1, D2 = 128, 32, 8
ED = 128
LANES = 16
CH = 640
M0, M1, M2 = 16383, 131071, 524287


def _sc_gather(xf, t0, t1, t2):
    n = xf.shape[0]
    info = plsc.get_sparse_core_info()
    nw = info.num_cores * info.num_subcores
    per_w = n // nw
    n_chunks = per_w // CH
    assert per_w % CH == 0 and n % nw == 0
    n4, n16 = n // 4, n // 16
    w_per_q, w_per_g = nw // 4, nw // 16

    mesh = plsc.VectorSubcoreMesh(core_axis_name="c", subcore_axis_name="s")

    @functools.partial(
        pl.kernel,
        out_type=(
            jax.ShapeDtypeStruct((n, D0), jnp.float32),
            jax.ShapeDtypeStruct((n4, 128), jnp.float32),
            jax.ShapeDtypeStruct((n16, 128), jnp.float32),
        ),
        mesh=mesh,
        compiler_params=pltpu.CompilerParams(use_tc_tiling_on_sc=False),
        scratch_types=[
            pltpu.VMEM((CH,), jnp.int32),
            pltpu.VMEM((CH,), jnp.int32),
            pltpu.VMEM((CH,), jnp.int32),
            pltpu.VMEM((CH,), jnp.int32),
            pltpu.VMEM((CH, D0), jnp.float32),
            pltpu.VMEM((CH, D1), jnp.float32),
            pltpu.VMEM((CH, D2), jnp.float32),
            pltpu.SemaphoreType.DMA,
            pltpu.SemaphoreType.DMA,
            pltpu.SemaphoreType.DMA,
        ],
    )
    def sc_kernel(x_hbm, t0_hbm, t1_hbm, t2_hbm, r0_hbm, r1p_hbm, r2p_hbm,
                  x_v, i0_v, i1_v, i2_v, r0_v, r1_v, r2_v, sem0, sem1, sem2):
        wid = lax.axis_index("s") * info.num_cores + lax.axis_index("c")
        w_base = wid * per_w
        q, g = wid // w_per_q, wid // w_per_g
        r1_row0 = (wid % w_per_q) * per_w
        r2_row0 = (wid % w_per_g) * per_w
        c1_off, c2_off = 32 * q, 8 * g

        def body(j, carry):
            base = w_base + j * CH
            pltpu.sync_copy(x_hbm.at[pl.ds(base, CH)], x_v)
            for i in range(CH // LANES):
                sl = pl.ds(i * LANES, LANES)
                xv = x_v[sl]
                zeros = jnp.zeros_like(xv)
                pv = base + i * LANES + lax.iota(jnp.int32, 16)
                i0_v[sl] = jnp.where(xv < CUT0, xv, pv & M0)
                d1 = pv & M1
                i1_v[sl] = jnp.where(
                    xv >= CUT0, jnp.where(xv < CUT1, xv - CUT0, d1), d1)
                i2_v[sl] = jnp.where(xv >= CUT1, xv - CUT1, pv & M2)
            c0 = pltpu.async_copy(t0_hbm.at[i0_v], r0_v, sem0)
            c1 = pltpu.async_copy(t1_hbm.at[i1_v], r1_v, sem1)
            c2 = pltpu.async_copy(t2_hbm.at[i2_v], r2_v, sem2)
            c0.wait()
            c1.wait()
            c2.wait()
            pltpu.sync_copy(r0_v, r0_hbm.at[pl.ds(base, CH)])
            pltpu.sync_copy(
                r1_v, r1p_hbm.at[pl.ds(r1_row0 + j * CH, CH), pl.ds(c1_off, D1)])
            pltpu.sync_copy(
                r2_v, r2p_hbm.at[pl.ds(r2_row0 + j * CH, CH), pl.ds(c2_off, D2)])
            return carry

        lax.fori_loop(0, n_chunks, body, 0)

    return sc_kernel(xf, t0, t1, t2)


def _tc_project(cid, r0, r1p, r2p, w0, w1t, w2t):
    n = r0.shape[0]
    bm = 512
    nb1 = (n // 4) // bm
    nb2 = (n // 16) // bm

    def body(cb, r0b, r1b, r2b, w0b, w1b, w2b, ob):
        q = pl.program_id(1)
        g = 4 * q + pl.program_id(0) // nb2
        riota = lax.broadcasted_iota(jnp.int32, (128, 1), 0)
        w1sel = jnp.where((riota >= D1 * q) & (riota < D1 * q + D1),
                          w1b[...], 0.0)
        w2sel = jnp.where((riota >= D2 * g) & (riota < D2 * g + D2),
                          w2b[...], 0.0)
        e0 = jnp.dot(r0b[...], w0b[...], preferred_element_type=jnp.float32)
        e1 = jnp.dot(r1b[...], w1sel, preferred_element_type=jnp.float32)
        e2 = jnp.dot(r2b[...], w2sel, preferred_element_type=jnp.float32)
        cv = cb[...].astype(jnp.int32)
        ob[...] = jnp.where(cv == 0, e0, jnp.where(cv == 1, e1, e2))

    return pl.pallas_call(
        body,
        grid=(nb1, 4),
        in_specs=[
            pl.BlockSpec((bm, 1), lambda i, j: (j * nb1 + i, 0)),
            pl.BlockSpec((bm, D0), lambda i, j: (j * nb1 + i, 0)),
            pl.BlockSpec((bm, 128), lambda i, j: (i, 0)),
            pl.BlockSpec((bm, 128), lambda i, j: (i % nb2, 0)),
            pl.BlockSpec((D0, ED), lambda i, j: (0, 0)),
            pl.BlockSpec((D0, ED), lambda i, j: (0, 0)),
            pl.BlockSpec((D0, ED), lambda i, j: (0, 0)),
        ],
        out_specs=pl.BlockSpec((bm, ED), lambda i, j: (j * nb1 + i, 0)),
        out_shape=jax.ShapeDtypeStruct((n, ED), jnp.float32),
    )(cid, r0, r1p, r2p, w0, w1t, w2t)


def kernel(x, t0, t1, t2, w0, w1, w2):
    b, s = x.shape
    n = b * s
    xf = x.reshape(n)
    cid = ((xf >= CUT0).astype(jnp.int8) + (xf >= CUT1).astype(jnp.int8))
    r0, r1p, r2p = _sc_gather(xf, t0, t1, t2)
    w1t = jnp.tile(w1, (4, 1))
    w2t = jnp.tile(w2, (16, 1))
    out = _tc_project(cid.reshape(n, 1), r0, r1p, r2p, w0, w1t, w2t)
    return out.reshape(b, s, ED)


# speedup vs baseline: 2.6174x; 1.2554x over previous
import functools

import jax
import jax.numpy as jnp
from jax import lax
from jax.experimental import pallas as pl
from jax.experimental.pallas import tpu as pltpu
from jax.experimental.pallas import tpu_sc as plsc

CUT0, CUT1 = 20000, 200000
D0, D1, D2 = 128, 32, 8
ED = 128
LANES = 16
CH = 640
M0, M1, M2 = 16383, 131071, 524287


def _sc_gather(xf, t0, t1, t2):
    n = xf.shape[0]
    info = plsc.get_sparse_core_info()
    nw = info.num_cores * info.num_subcores
    per_w = n // nw
    n_chunks = per_w // CH
    assert per_w % CH == 0 and n % nw == 0
    n4, n16 = n // 4, n // 16
    w_per_q, w_per_g = nw // 4, nw // 16

    mesh = plsc.VectorSubcoreMesh(core_axis_name="c", subcore_axis_name="s")

    @functools.partial(
        pl.kernel,
        out_type=(
            jax.ShapeDtypeStruct((n, D0), jnp.float32),
            jax.ShapeDtypeStruct((n4, 128), jnp.float32),
            jax.ShapeDtypeStruct((n16, 128), jnp.float32),
        ),
        mesh=mesh,
        compiler_params=pltpu.CompilerParams(use_tc_tiling_on_sc=False),
        scratch_types=[
            pltpu.VMEM((CH,), jnp.int32),
            pltpu.VMEM((CH,), jnp.int32),
            pltpu.VMEM((CH,), jnp.int32),
            pltpu.VMEM((CH,), jnp.int32),
            pltpu.VMEM((CH, D0), jnp.float32),
            pltpu.VMEM((CH, D1), jnp.float32),
            pltpu.VMEM((CH, D2), jnp.float32),
            pltpu.SemaphoreType.DMA,
            pltpu.SemaphoreType.DMA,
            pltpu.SemaphoreType.DMA,
        ],
    )
    def sc_kernel(x_hbm, t0_hbm, t1_hbm, t2_hbm, r0_hbm, r1p_hbm, r2p_hbm,
                  x_v, i0_v, i1_v, i2_v, r0_v, r1_v, r2_v, sem0, sem1, sem2):
        wid = lax.axis_index("s") * info.num_cores + lax.axis_index("c")
        w_base = wid * per_w
        q, g = wid // w_per_q, wid // w_per_g
        r1_row0 = (wid % w_per_q) * per_w
        r2_row0 = (wid % w_per_g) * per_w
        c1_off, c2_off = 32 * q, 8 * g

        def body(j, carry):
            base = w_base + j * CH
            pltpu.sync_copy(x_hbm.at[pl.ds(base, CH)], x_v)
            for i in range(CH // LANES):
                sl = pl.ds(i * LANES, LANES)
                xv = x_v[sl]
                zeros = jnp.zeros_like(xv)
                pv = base + i * LANES + lax.iota(jnp.int32, 16)
                i0_v[sl] = jnp.where(xv < CUT0, xv, pv & M0)
                d1 = pv & M1
                i1_v[sl] = jnp.where(
                    xv >= CUT0, jnp.where(xv < CUT1, xv - CUT0, d1), d1)
                i2_v[sl] = jnp.where(xv >= CUT1, xv - CUT1, pv & M2)
            c0 = pltpu.async_copy(t0_hbm.at[i0_v], r0_v, sem0)
            c1 = pltpu.async_copy(t1_hbm.at[i1_v], r1_v, sem1)
            c2 = pltpu.async_copy(t2_hbm.at[i2_v], r2_v, sem2)
            c0.wait()
            c1.wait()
            c2.wait()
            pltpu.sync_copy(r0_v, r0_hbm.at[pl.ds(base, CH)])
            pltpu.sync_copy(
                r1_v, r1p_hbm.at[pl.ds(r1_row0 + j * CH, CH), pl.ds(c1_off, D1)])
            pltpu.sync_copy(
                r2_v, r2p_hbm.at[pl.ds(r2_row0 + j * CH, CH), pl.ds(c2_off, D2)])
            return carry

        lax.fori_loop(0, n_chunks, body, 0)

    return sc_kernel(xf, t0, t1, t2)


def _tc_project(cid, r0, r1p, r2p, w0, w1t, w2t):
    n = r0.shape[0]
    bm = 2560
    nb1 = (n // 4) // bm
    nb2 = (n // 16) // bm

    def body(cb, r0b, r1b, r2b, w0b, w1b, w2b, ob):
        q = pl.program_id(1)
        g = 4 * q + pl.program_id(0) // nb2
        riota = lax.broadcasted_iota(jnp.int32, (128, 1), 0)
        w1sel = jnp.where((riota >= D1 * q) & (riota < D1 * q + D1),
                          w1b[...], 0.0)
        w2sel = jnp.where((riota >= D2 * g) & (riota < D2 * g + D2),
                          w2b[...], 0.0)
        e0 = jnp.dot(r0b[...], w0b[...], preferred_element_type=jnp.float32)
        e1 = jnp.dot(r1b[...], w1sel, preferred_element_type=jnp.float32)
        e2 = jnp.dot(r2b[...], w2sel, preferred_element_type=jnp.float32)
        cv = cb[...].astype(jnp.int32)
        ob[...] = jnp.where(cv == 0, e0, jnp.where(cv == 1, e1, e2))

    return pl.pallas_call(
        body,
        grid=(nb1, 4),
        in_specs=[
            pl.BlockSpec((bm, 1), lambda i, j: (j * nb1 + i, 0)),
            pl.BlockSpec((bm, D0), lambda i, j: (j * nb1 + i, 0)),
            pl.BlockSpec((bm, 128), lambda i, j: (i, 0)),
            pl.BlockSpec((bm, 128), lambda i, j: (i % nb2, 0)),
            pl.BlockSpec((D0, ED), lambda i, j: (0, 0)),
            pl.BlockSpec((D0, ED), lambda i, j: (0, 0)),
            pl.BlockSpec((D0, ED), lambda i, j: (0, 0)),
        ],
        out_specs=pl.BlockSpec((bm, ED), lambda i, j: (j * nb1 + i, 0)),
        out_shape=jax.ShapeDtypeStruct((n, ED), jnp.float32),
    )(cid, r0, r1p, r2p, w0, w1t, w2t)


def kernel(x, t0, t1, t2, w0, w1, w2):
    b, s = x.shape
    n = b * s
    xf = x.reshape(n)
    cid = ((xf >= CUT0).astype(jnp.int8) + (xf >= CUT1).astype(jnp.int8))
    r0, r1p, r2p = _sc_gather(xf, t0, t1, t2)
    w1t = jnp.tile(w1, (4, 1))
    w2t = jnp.tile(w2, (16, 1))
    out = _tc_project(cid.reshape(n, 1), r0, r1p, r2p, w0, w1t, w2t)
    return out.reshape(b, s, ED)


# speedup vs baseline: 2.7404x; 1.0470x over previous
import functools

import jax
import jax.numpy as jnp
from jax import lax
from jax.experimental import pallas as pl
from jax.experimental.pallas import tpu as pltpu
from jax.experimental.pallas import tpu_sc as plsc

CUT0, CUT1 = 20000, 200000
D0, D1, D2 = 128, 32, 8
ED = 128
LANES = 16
CH = 640
M0, M1, M2 = 16383, 131071, 524287


def _sc_gather(xf, t0, t1, t2):
    n = xf.shape[0]
    info = plsc.get_sparse_core_info()
    nw = info.num_cores * info.num_subcores
    per_w = n // nw
    n_chunks = per_w // CH
    assert per_w % CH == 0 and n % nw == 0
    n4, n16 = n // 4, n // 16
    w_per_q, w_per_g = nw // 4, nw // 16

    mesh = plsc.VectorSubcoreMesh(core_axis_name="c", subcore_axis_name="s")

    @functools.partial(
        pl.kernel,
        out_type=(
            jax.ShapeDtypeStruct((n, D0), jnp.float32),
            jax.ShapeDtypeStruct((n4, 128), jnp.float32),
            jax.ShapeDtypeStruct((n16, 128), jnp.float32),
        ),
        mesh=mesh,
        compiler_params=pltpu.CompilerParams(use_tc_tiling_on_sc=False),
        scratch_types=[
            pltpu.VMEM((CH,), jnp.int32),
            pltpu.VMEM((CH,), jnp.int32),
            pltpu.VMEM((CH,), jnp.int32),
            pltpu.VMEM((CH,), jnp.int32),
            pltpu.VMEM((CH, D0), jnp.float32),
            pltpu.VMEM((CH, D1), jnp.float32),
            pltpu.VMEM((CH, D2), jnp.float32),
            pltpu.SemaphoreType.DMA,
            pltpu.SemaphoreType.DMA,
            pltpu.SemaphoreType.DMA,
        ],
    )
    def sc_kernel(x_hbm, t0_hbm, t1_hbm, t2_hbm, r0_hbm, r1p_hbm, r2p_hbm,
                  x_v, i0_v, i1_v, i2_v, r0_v, r1_v, r2_v, sem0, sem1, sem2):
        wid = lax.axis_index("s") * info.num_cores + lax.axis_index("c")
        w_base = wid * per_w
        q, g = wid // w_per_q, wid // w_per_g
        r1_row0 = (wid % w_per_q) * per_w
        r2_row0 = (wid % w_per_g) * per_w
        c1_off, c2_off = 32 * q, 8 * g

        def body(j, carry):
            base = w_base + j * CH
            pltpu.sync_copy(x_hbm.at[pl.ds(base, CH)], x_v)
            for i in range(CH // LANES):
                sl = pl.ds(i * LANES, LANES)
                xv = x_v[sl]
                zeros = jnp.zeros_like(xv)
                pv = base + i * LANES + lax.iota(jnp.int32, 16)
                i0_v[sl] = jnp.where(xv < CUT0, xv, pv & M0)
                d1 = pv & M1
                i1_v[sl] = jnp.where(
                    xv >= CUT0, jnp.where(xv < CUT1, xv - CUT0, d1), d1)
                i2_v[sl] = jnp.where(xv >= CUT1, xv - CUT1, pv & M2)
            c0 = pltpu.async_copy(t0_hbm.at[i0_v], r0_v, sem0)
            c1 = pltpu.async_copy(t1_hbm.at[i1_v], r1_v, sem1)
            c2 = pltpu.async_copy(t2_hbm.at[i2_v], r2_v, sem2)
            c0.wait()
            c1.wait()
            c2.wait()
            pltpu.sync_copy(r0_v, r0_hbm.at[pl.ds(base, CH)])
            pltpu.sync_copy(
                r1_v, r1p_hbm.at[pl.ds(r1_row0 + j * CH, CH), pl.ds(c1_off, D1)])
            pltpu.sync_copy(
                r2_v, r2p_hbm.at[pl.ds(r2_row0 + j * CH, CH), pl.ds(c2_off, D2)])
            return carry

        lax.fori_loop(0, n_chunks, body, 0)

    return sc_kernel(xf, t0, t1, t2)


def _tc_project(cid, r0, r1p, r2p, w0, w1t, w2t):
    n = r0.shape[0]
    bm = 6400
    nb1 = (n // 4) // bm
    nb2 = (n // 16) // bm

    def body(cb, r0b, r1b, r2b, w0b, w1b, w2b, ob):
        q = pl.program_id(1)
        g = 4 * q + pl.program_id(0) // nb2
        riota = lax.broadcasted_iota(jnp.int32, (128, 1), 0)
        w1sel = jnp.where((riota >= D1 * q) & (riota < D1 * q + D1),
                          w1b[...], 0.0)
        w2sel = jnp.where((riota >= D2 * g) & (riota < D2 * g + D2),
                          w2b[...], 0.0)
        e0 = jnp.dot(r0b[...], w0b[...], preferred_element_type=jnp.float32)
        e1 = jnp.dot(r1b[...], w1sel, preferred_element_type=jnp.float32)
        e2 = jnp.dot(r2b[...], w2sel, preferred_element_type=jnp.float32)
        cv = cb[...].astype(jnp.int32)
        ob[...] = jnp.where(cv == 0, e0, jnp.where(cv == 1, e1, e2))

    return pl.pallas_call(
        body,
        grid=(nb1, 4),
        in_specs=[
            pl.BlockSpec((bm, 1), lambda i, j: (j * nb1 + i, 0)),
            pl.BlockSpec((bm, D0), lambda i, j: (j * nb1 + i, 0)),
            pl.BlockSpec((bm, 128), lambda i, j: (i, 0)),
            pl.BlockSpec((bm, 128), lambda i, j: (i % nb2, 0)),
            pl.BlockSpec((D0, ED), lambda i, j: (0, 0)),
            pl.BlockSpec((D0, ED), lambda i, j: (0, 0)),
            pl.BlockSpec((D0, ED), lambda i, j: (0, 0)),
        ],
        out_specs=pl.BlockSpec((bm, ED), lambda i, j: (j * nb1 + i, 0)),
        out_shape=jax.ShapeDtypeStruct((n, ED), jnp.float32),
    )(cid, r0, r1p, r2p, w0, w1t, w2t)


def kernel(x, t0, t1, t2, w0, w1, w2):
    b, s = x.shape
    n = b * s
    xf = x.reshape(n)
    cid = ((xf >= CUT0).astype(jnp.int8) + (xf >= CUT1).astype(jnp.int8))
    r0, r1p, r2p = _sc_gather(xf, t0, t1, t2)
    w1t = jnp.tile(w1, (4, 1))
    w2t = jnp.tile(w2, (16, 1))
    out = _tc_project(cid.reshape(n, 1), r0, r1p, r2p, w0, w1t, w2t)
    return out.reshape(b, s, ED)
